# Initial kernel scaffold; baseline (speedup 1.0000x reference)
import jax
import jax.numpy as jnp
from jax.experimental import pallas as pl


def kernel(bos_tensor, table):
    raise NotImplementedError("write your pallas kernel here")


# speedup vs baseline: 5.7963x; 5.7963x over previous
import functools

import jax
import jax.numpy as jnp
from jax import lax
from jax.experimental import pallas as pl
from jax.experimental.pallas import tpu as pltpu
from jax.experimental.pallas import tpu_sc as plsc

DIM = 64
B_ROWS = 16384 * 50
NUM_WORKERS = 32
B_PER_W = B_ROWS // NUM_WORKERS
CHUNK = 512
N_CHUNKS = B_PER_W // CHUNK


def _sc_gather(table, idx_flat):
    mesh = plsc.VectorSubcoreMesh(core_axis_name="c", subcore_axis_name="s")

    @functools.partial(
        pl.kernel,
        mesh=mesh,
        compiler_params=pltpu.CompilerParams(use_tc_tiling_on_sc=False),
        out_type=jax.ShapeDtypeStruct((B_ROWS, DIM), jnp.float32),
        scratch_types=[
            pltpu.VMEM((CHUNK,), jnp.int32),
            pltpu.VMEM((CHUNK, DIM), jnp.float32),
            pltpu.SemaphoreType.DMA,
        ],
    )
    def k(table_hbm, idx_hbm, out_hbm, idx_v, rows_v, sem):
        wid = lax.axis_index("s") * 2 + lax.axis_index("c")
        base = wid * B_PER_W

        def body(i, carry):
            off = base + i * CHUNK
            pltpu.sync_copy(idx_hbm.at[pl.ds(off, CHUNK)], idx_v)
            pltpu.async_copy(table_hbm.at[idx_v], rows_v, sem).wait()
            pltpu.sync_copy(rows_v, out_hbm.at[pl.ds(off, CHUNK)])
            return carry

        lax.fori_loop(0, N_CHUNKS, body, 0)

    return k(table, idx_flat)


def kernel(bos_tensor, table):
    idx = bos_tensor.reshape(-1).astype(jnp.int32)
    out = _sc_gather(table, idx)
    return out.reshape(bos_tensor.shape[0], bos_tensor.shape[1], DIM)


# speedup vs baseline: 6.2030x; 1.0702x over previous
import functools

import jax
import jax.numpy as jnp
from jax import lax
from jax.experimental import pallas as pl
from jax.experimental.pallas import tpu as pltpu
from jax.experimental.pallas import tpu_sc as plsc

DIM = 64
B_ROWS = 16384 * 50
NUM_WORKERS = 32
B_PER_W = B_ROWS // NUM_WORKERS
CHUNK = 256
N_CHUNKS = B_PER_W // CHUNK
NBUF = 4
N_GROUPS = N_CHUNKS // NBUF


def _sc_gather(table, idx_flat):
    mesh = plsc.VectorSubcoreMesh(core_axis_name="c", subcore_axis_name="s")

    @functools.partial(
        pl.kernel,
        mesh=mesh,
        compiler_params=pltpu.CompilerParams(use_tc_tiling_on_sc=False),
        out_type=jax.ShapeDtypeStruct((B_ROWS, DIM), jnp.float32),
        scratch_types=[
            pltpu.VMEM((B_PER_W,), jnp.int32),
            pltpu.VMEM((NBUF, CHUNK, DIM), jnp.float32),
            pltpu.SemaphoreType.DMA,
            pltpu.SemaphoreType.DMA,
            pltpu.SemaphoreType.DMA,
            pltpu.SemaphoreType.DMA,
            pltpu.SemaphoreType.DMA,
            pltpu.SemaphoreType.DMA,
            pltpu.SemaphoreType.DMA,
            pltpu.SemaphoreType.DMA,
        ],
    )
    def k(table_hbm, idx_hbm, out_hbm, idx_v, rows_v,
          sg0, sg1, sg2, sg3, so0, so1, so2, so3):
        semg = (sg0, sg1, sg2, sg3)
        semo = (so0, so1, so2, so3)
        wid = lax.axis_index("s") * 2 + lax.axis_index("c")
        base = wid * B_PER_W

        pltpu.sync_copy(idx_hbm.at[pl.ds(base, B_PER_W)], idx_v)

        def start_gather(i, b):
            src = table_hbm.at[idx_v.at[pl.ds(i * CHUNK, CHUNK)]]
            pltpu.async_copy(src, rows_v.at[b], semg[b])

        def wait_gather(b):
            pltpu.make_async_copy(
                table_hbm.at[pl.ds(0, CHUNK)], rows_v.at[b], semg[b]).wait()

        def start_out(i, b):
            pltpu.async_copy(
                rows_v.at[b], out_hbm.at[pl.ds(base + i * CHUNK, CHUNK)],
                semo[b])

        def wait_out(i, b):
            pltpu.make_async_copy(
                rows_v.at[b], out_hbm.at[pl.ds(base + i * CHUNK, CHUNK)],
                semo[b]).wait()

        for b in range(NBUF):
            start_gather(b, b)

        def body(j, carry):
            i0 = j * NBUF
            for b in range(NBUF):
                wait_gather(b)
                start_out(i0 + b, b)
            for b in range(NBUF):
                wait_out(i0 + b, b)
                start_gather(i0 + NBUF + b, b)
            return carry

        lax.fori_loop(0, N_GROUPS - 1, body, 0)

        i0 = (N_GROUPS - 1) * NBUF
        for b in range(NBUF):
            wait_gather(b)
            start_out(i0 + b, b)
        for b in range(NBUF):
            wait_out(i0 + b, b)

    return k(table, idx_flat)


def kernel(bos_tensor, table):
    idx = bos_tensor.reshape(-1).astype(jnp.int32)
    out = _sc_gather(table, idx)
    return out.reshape(bos_tensor.shape[0], bos_tensor.shape[1], DIM)
